# baseline (speedup 1.0000x reference)
import functools

import jax
import jax.numpy as jnp
from jax import lax
from jax.experimental import pallas as pl
from jax.experimental.pallas import tpu as pltpu
from jax.experimental.pallas import tpu_sc as plsc

_K = 2048
_NROWS = 128
_NCOLS = 32768
_NWORKERS = 32
_ROWS_PER_WORKER = _NROWS // _NWORKERS
_NSLICES = _NCOLS // 16
_UNROLL = 16


@functools.partial(
    pl.kernel,
    out_type=jax.ShapeDtypeStruct((_NROWS, 16), jnp.float32),
    scratch_types=[
        pltpu.VMEM((_NCOLS,), jnp.float32),
        pltpu.VMEM((_NCOLS,), jnp.int32),
        pltpu.VMEM((8736,), jnp.int32),
        pltpu.VMEM((256,), jnp.int32),
        pltpu.VMEM((16,), jnp.float32),
    ],
    mesh=plsc.VectorSubcoreMesh(core_axis_name="c", subcore_axis_name="s"),
    compiler_params=pltpu.CompilerParams(needs_layout_passes=False),
)
def _sc_thresholds(x_hbm, thr_hbm, row_v, key_v, hist_v, htot_v, thr_v):
    wid = lax.axis_index("s") * 2 + lax.axis_index("c")
    lane = lax.iota(jnp.int32, 16)
    lane_base = lane * 273
    ones16 = jnp.ones((16,), jnp.int32)
    zeros16 = jnp.zeros((16,), jnp.int32)

    def suffix(v):
        return lax.rev(jnp.cumsum(lax.rev(v, (0,))), (0,))

    def splat_count(mask):
        return jnp.max(plsc.all_reduce_population_count(mask))

    def do_row(r, _):
        row = wid * _ROWS_PER_WORKER + r
        pltpu.sync_copy(x_hbm.at[row], row_v)

        prefix = jnp.int32(0)
        krem = jnp.int32(_K)
        for p in range(4):
            shift = 24 - 8 * p

            @plsc.parallel_loop(0, 546, 1, unroll=3)
            def _(j):
                hist_v[pl.ds(j * 16, 16)] = zeros16

            if p == 0:
                @plsc.parallel_loop(0, _NSLICES, 1, unroll=_UNROLL)
                def _(i):
                    s = i * 16
                    copy = (i & 1) * 4368
                    xv = row_v[pl.ds(s, 16)]
                    b = lax.bitcast_convert_type(xv, jnp.int32)
                    m = lax.shift_right_arithmetic(b, 31)
                    key = b ^ (m | jnp.int32(-(2**31)))
                    key_v[pl.ds(s, 16)] = key
                    digit = lax.shift_right_logical(key, 24)
                    plsc.addupdate_scatter(
                        hist_v, [lane_base + digit + copy], ones16)
            else:
                pfx = prefix

                @plsc.parallel_loop(0, _NSLICES, 1, unroll=_UNROLL)
                def _(i):
                    s = i * 16
                    copy = (i & 1) * 4368
                    key = key_v[pl.ds(s, 16)]
                    act = lax.shift_right_logical(key, shift + 8) == pfx
                    digit = lax.shift_right_logical(key, shift) & 255
                    plsc.addupdate_scatter(
                        hist_v, [lane_base + digit + copy], ones16, mask=act)

            gt = zeros16
            for j in range(16):
                parts = [hist_v[pl.ds(c * 4368 + l * 273 + j * 16, 16)]
                         for l in range(16) for c in (0, 1)]
                while len(parts) > 1:
                    parts = [a + b for a, b in zip(parts[::2], parts[1::2])]
                acc = parts[0]
                htot_v[pl.ds(j * 16, 16)] = acc
                gt = gt + jnp.where(lane == j, jnp.sum(acc), 0)
            jstar = splat_count(suffix(gt) >= krem) - 1
            hj = htot_v[pl.ds(jstar * 16, 16)]
            excl = jnp.sum(jnp.where(lane > jstar, gt, 0))
            sfx_w = suffix(hj) + excl
            lstar = splat_count(sfx_w >= krem) - 1
            s_d = excl + jnp.sum(jnp.where(lane >= lstar, hj, 0))
            h_d = jnp.sum(jnp.where(lane == lstar, hj, 0))
            krem = krem - (s_d - h_d)
            prefix = (prefix * 256) + jstar * 16 + lstar

        ukey = jnp.broadcast_to(prefix, (16,))
        thr_bits = jnp.where(ukey < 0, ukey ^ jnp.int32(-(2**31)), ~ukey)
        thr_v[...] = lax.bitcast_convert_type(thr_bits, jnp.float32)
        pltpu.sync_copy(thr_v, thr_hbm.at[row])
        return 0

    lax.fori_loop(0, _ROWS_PER_WORKER, do_row, 0)


def _mask_block(x_ref, t_ref, o_ref):
    x = x_ref[...]
    thr = t_ref[:, 0:1]
    o_ref[...] = jnp.where(x >= thr, x, jnp.float32(0.0))


def _tc_mask(inputs, thr_bcast):
    r = 64
    return pl.pallas_call(
        _mask_block,
        grid=(_NROWS // r,),
        in_specs=[
            pl.BlockSpec((r, _NCOLS), lambda i: (i, 0)),
            pl.BlockSpec((r, 128), lambda i: (i, 0)),
        ],
        out_specs=pl.BlockSpec((r, _NCOLS), lambda i: (i, 0)),
        out_shape=jax.ShapeDtypeStruct((_NROWS, _NCOLS), jnp.float32),
    )(inputs, thr_bcast)


def kernel(inputs, k):
    del k
    thr = _sc_thresholds(inputs)
    thr_bcast = jnp.broadcast_to(thr[:, 0:1], (_NROWS, 128))
    return _tc_mask(inputs, thr_bcast)


# speedup vs baseline: 1.2620x; 1.2620x over previous
import functools

import jax
import jax.numpy as jnp
from jax import lax
from jax.experimental import pallas as pl
from jax.experimental.pallas import tpu as pltpu
from jax.experimental.pallas import tpu_sc as plsc

_K = 2048
_NROWS = 128
_NCOLS = 32768
_NWORKERS = 32
_ROWS_PER_WORKER = _NROWS // _NWORKERS
_NSLICES = _NCOLS // 16
_UNROLL = 8


@functools.partial(
    pl.kernel,
    out_type=jax.ShapeDtypeStruct((_NROWS, 16), jnp.float32),
    scratch_types=[
        pltpu.VMEM((_NCOLS,), jnp.float32),
        pltpu.VMEM((_NCOLS,), jnp.float32),
        pltpu.VMEM((4368,), jnp.int32),
        pltpu.VMEM((256,), jnp.int32),
        pltpu.VMEM((16,), jnp.float32),
        pltpu.SemaphoreType.DMA,
        pltpu.SemaphoreType.DMA,
    ],
    mesh=plsc.VectorSubcoreMesh(core_axis_name="c", subcore_axis_name="s"),
    compiler_params=pltpu.CompilerParams(needs_layout_passes=False),
)
def _sc_thresholds(x_hbm, thr_hbm, row_a, row_b, hist_v, htot_v,
                   thr_v, sem_a, sem_b):
    wid = lax.axis_index("s") * 2 + lax.axis_index("c")
    lane = lax.iota(jnp.int32, 16)
    lane_base = lane * 273
    ones16 = jnp.ones((16,), jnp.int32)
    zeros16 = jnp.zeros((16,), jnp.int32)

    def suffix(v):
        return lax.rev(jnp.cumsum(lax.rev(v, (0,))), (0,))

    def splat_count(mask):
        return jnp.max(plsc.all_reduce_population_count(mask))

    base = wid * _ROWS_PER_WORKER
    bufs = [row_a, row_b]
    sems = [sem_a, sem_b]
    handles = [None] * _ROWS_PER_WORKER
    handles[0] = pltpu.async_copy(x_hbm.at[base], row_a, sem_a)
    for r in range(_ROWS_PER_WORKER):
        row = base + r
        handles[r].wait()
        if r + 1 < _ROWS_PER_WORKER:
            handles[r + 1] = pltpu.async_copy(
                x_hbm.at[row + 1], bufs[(r + 1) % 2], sems[(r + 1) % 2])
        row_v = bufs[r % 2]

        prefix = jnp.int32(0)
        krem = jnp.int32(_K)
        for p in range(4):
            shift = 24 - 8 * p

            @plsc.parallel_loop(0, 273, 1, unroll=3)
            def _(j):
                hist_v[pl.ds(j * 16, 16)] = zeros16

            pfx = prefix

            @plsc.parallel_loop(0, _NSLICES, 1, unroll=_UNROLL)
            def _(i):
                s = i * 16
                xv = row_v[pl.ds(s, 16)]
                b = lax.bitcast_convert_type(xv, jnp.int32)
                m = lax.shift_right_arithmetic(b, 31)
                key = b ^ (m | jnp.int32(-(2**31)))
                digit = lax.shift_right_logical(key, shift) & 255
                if p == 0:
                    plsc.addupdate_scatter(
                        hist_v, [lane_base + digit], ones16)
                else:
                    act = lax.shift_right_logical(key, shift + 8) == pfx
                    plsc.addupdate_scatter(
                        hist_v, [lane_base + digit], ones16, mask=act)

            def merge(j, gt):
                parts = [hist_v[pl.ds(l * 273 + j * 16, 16)] for l in range(16)]
                while len(parts) > 1:
                    parts = [a + b for a, b in zip(parts[::2], parts[1::2])]
                acc = parts[0]
                htot_v[pl.ds(j * 16, 16)] = acc
                return gt + jnp.where(lane == j, jnp.sum(acc), 0)

            gt = lax.fori_loop(0, 16, merge, zeros16)
            jstar = splat_count(suffix(gt) >= krem) - 1
            hj = htot_v[pl.ds(jstar * 16, 16)]
            excl = jnp.sum(jnp.where(lane > jstar, gt, 0))
            sfx_w = suffix(hj) + excl
            lstar = splat_count(sfx_w >= krem) - 1
            s_d = excl + jnp.sum(jnp.where(lane >= lstar, hj, 0))
            h_d = jnp.sum(jnp.where(lane == lstar, hj, 0))
            krem = krem - (s_d - h_d)
            prefix = (prefix * 256) + jstar * 16 + lstar

        ukey = jnp.broadcast_to(prefix, (16,))
        thr_bits = jnp.where(ukey < 0, ukey ^ jnp.int32(-(2**31)), ~ukey)
        thr_v[...] = lax.bitcast_convert_type(thr_bits, jnp.float32)
        pltpu.sync_copy(thr_v, thr_hbm.at[row])


def _mask_block(x_ref, t_ref, o_ref):
    x = x_ref[...]
    thr = t_ref[:, 0:1]
    o_ref[...] = jnp.where(x >= thr, x, jnp.float32(0.0))


def _tc_mask(inputs, thr_bcast):
    r = 64
    return pl.pallas_call(
        _mask_block,
        grid=(_NROWS // r,),
        in_specs=[
            pl.BlockSpec((r, _NCOLS), lambda i: (i, 0)),
            pl.BlockSpec((r, 128), lambda i: (i, 0)),
        ],
        out_specs=pl.BlockSpec((r, _NCOLS), lambda i: (i, 0)),
        out_shape=jax.ShapeDtypeStruct((_NROWS, _NCOLS), jnp.float32),
    )(inputs, thr_bcast)


def kernel(inputs, k):
    del k
    thr = _sc_thresholds(inputs)
    thr_bcast = jnp.broadcast_to(thr[:, 0:1], (_NROWS, 128))
    return _tc_mask(inputs, thr_bcast)


# speedup vs baseline: 1.3893x; 1.1009x over previous
import functools

import jax
import jax.numpy as jnp
from jax import lax
from jax.experimental import pallas as pl
from jax.experimental.pallas import tpu as pltpu
from jax.experimental.pallas import tpu_sc as plsc

_K = 2048
_NROWS = 128
_NCOLS = 32768
_NWORKERS = 32
_ROWS_PER_WORKER = _NROWS // _NWORKERS
_NSLICES = _NCOLS // 16
_UNROLL = 8


@functools.partial(
    pl.kernel,
    out_type=jax.ShapeDtypeStruct((_NROWS, 16), jnp.float32),
    scratch_types=[
        pltpu.VMEM((_NCOLS,), jnp.float32),
        pltpu.VMEM((_NCOLS,), jnp.float32),
        pltpu.VMEM((_NCOLS + 176,), jnp.int32),
        pltpu.VMEM((4368,), jnp.int32),
        pltpu.VMEM((256,), jnp.int32),
        pltpu.VMEM((16,), jnp.float32),
        pltpu.SemaphoreType.DMA,
        pltpu.SemaphoreType.DMA,
    ],
    mesh=plsc.VectorSubcoreMesh(core_axis_name="c", subcore_axis_name="s"),
    compiler_params=pltpu.CompilerParams(needs_layout_passes=False),
)
def _sc_thresholds(x_hbm, thr_hbm, row_a, row_b, cand_v, hist_v, htot_v,
                   thr_v, sem_a, sem_b):
    wid = lax.axis_index("s") * 2 + lax.axis_index("c")
    lane = lax.iota(jnp.int32, 16)
    lane_base = lane * 273
    ones16 = jnp.ones((16,), jnp.int32)
    zeros16 = jnp.zeros((16,), jnp.int32)

    def suffix(v):
        return lax.rev(jnp.cumsum(lax.rev(v, (0,))), (0,))

    def splat_count(mask):
        return jnp.max(plsc.all_reduce_population_count(mask))

    base = wid * _ROWS_PER_WORKER
    bufs = [row_a, row_b]
    sems = [sem_a, sem_b]
    handles = [None] * _ROWS_PER_WORKER
    handles[0] = pltpu.async_copy(x_hbm.at[base], row_a, sem_a)
    for r in range(_ROWS_PER_WORKER):
        row = base + r
        handles[r].wait()
        if r + 1 < _ROWS_PER_WORKER:
            handles[r + 1] = pltpu.async_copy(
                x_hbm.at[row + 1], bufs[(r + 1) % 2], sems[(r + 1) % 2])
        row_v = bufs[r % 2]

        prefix = jnp.int32(0)
        krem = jnp.int32(_K)
        ncand = jnp.int32(0)
        for p in range(4):
            shift = 24 - 8 * p

            @plsc.parallel_loop(0, 273, 1, unroll=3)
            def _(j):
                hist_v[pl.ds(j * 16, 16)] = zeros16

            pfx = prefix

            if p == 0:
                @plsc.parallel_loop(0, _NSLICES, 1, unroll=_UNROLL)
                def _(i):
                    s = i * 16
                    xv = row_v[pl.ds(s, 16)]
                    b = lax.bitcast_convert_type(xv, jnp.int32)
                    m = lax.shift_right_arithmetic(b, 31)
                    key = b ^ (m | jnp.int32(-(2**31)))
                    digit = lax.shift_right_logical(key, shift) & 255
                    plsc.addupdate_scatter(
                        hist_v, [lane_base + digit], ones16)
            elif p == 1:
                def body(i, off):
                    s = i * 16
                    xv = row_v[pl.ds(s, 16)]
                    b = lax.bitcast_convert_type(xv, jnp.int32)
                    m = lax.shift_right_arithmetic(b, 31)
                    key = b ^ (m | jnp.int32(-(2**31)))
                    act = lax.shift_right_logical(key, shift + 8) == pfx
                    digit = lax.shift_right_logical(key, shift) & 255
                    plsc.addupdate_scatter(
                        hist_v, [lane_base + digit], ones16, mask=act)
                    pos = off + jnp.cumsum(jnp.where(act, 1, 0)) - 1
                    plsc.store_scatter(cand_v, [pos], key, mask=act)
                    return off + plsc.all_reduce_population_count(act)

                off = plsc.parallel_loop(
                    0, _NSLICES, 1, unroll=_UNROLL, carry=zeros16)(body)
                ncand = jnp.max(off)
                sent = lax.shift_left(
                    jnp.broadcast_to(pfx ^ 128, (16,)), 24)
                for t in range(10):
                    plsc.store_scatter(
                        cand_v, [off + t * 16 + lane], sent)
            else:
                nsl = jnp.max(lax.shift_right_logical(off + 15, 4))

                @plsc.parallel_loop(0, nsl, 1, unroll=4)
                def _(i):
                    s = i * 16
                    key = cand_v[pl.ds(s, 16)]
                    act = lax.shift_right_logical(key, shift + 8) == pfx
                    digit = lax.shift_right_logical(key, shift) & 255
                    plsc.addupdate_scatter(
                        hist_v, [lane_base + digit], ones16, mask=act)

            def merge(j, gt):
                parts = [hist_v[pl.ds(l * 273 + j * 16, 16)] for l in range(16)]
                while len(parts) > 1:
                    parts = [a + b for a, b in zip(parts[::2], parts[1::2])]
                acc = parts[0]
                htot_v[pl.ds(j * 16, 16)] = acc
                return gt + jnp.where(lane == j, jnp.sum(acc), 0)

            gt = lax.fori_loop(0, 16, merge, zeros16)
            jstar = splat_count(suffix(gt) >= krem) - 1
            hj = htot_v[pl.ds(jstar * 16, 16)]
            excl = jnp.sum(jnp.where(lane > jstar, gt, 0))
            sfx_w = suffix(hj) + excl
            lstar = splat_count(sfx_w >= krem) - 1
            s_d = excl + jnp.sum(jnp.where(lane >= lstar, hj, 0))
            h_d = jnp.sum(jnp.where(lane == lstar, hj, 0))
            krem = krem - (s_d - h_d)
            prefix = (prefix * 256) + jstar * 16 + lstar

        ukey = jnp.broadcast_to(prefix, (16,))
        thr_bits = jnp.where(ukey < 0, ukey ^ jnp.int32(-(2**31)), ~ukey)
        thr_v[...] = lax.bitcast_convert_type(thr_bits, jnp.float32)
        pltpu.sync_copy(thr_v, thr_hbm.at[row])


def _mask_block(x_ref, t_ref, o_ref):
    x = x_ref[...]
    thr = t_ref[:, 0:1]
    o_ref[...] = jnp.where(x >= thr, x, jnp.float32(0.0))


def _tc_mask(inputs, thr_bcast):
    r = 64
    return pl.pallas_call(
        _mask_block,
        grid=(_NROWS // r,),
        in_specs=[
            pl.BlockSpec((r, _NCOLS), lambda i: (i, 0)),
            pl.BlockSpec((r, 128), lambda i: (i, 0)),
        ],
        out_specs=pl.BlockSpec((r, _NCOLS), lambda i: (i, 0)),
        out_shape=jax.ShapeDtypeStruct((_NROWS, _NCOLS), jnp.float32),
    )(inputs, thr_bcast)


def kernel(inputs, k):
    del k
    thr = _sc_thresholds(inputs)
    thr_bcast = jnp.broadcast_to(thr[:, 0:1], (_NROWS, 128))
    return _tc_mask(inputs, thr_bcast)


# speedup vs baseline: 1.4195x; 1.0217x over previous
import functools

import jax
import jax.numpy as jnp
from jax import lax
from jax.experimental import pallas as pl
from jax.experimental.pallas import tpu as pltpu
from jax.experimental.pallas import tpu_sc as plsc

_K = 2048
_NROWS = 128
_NCOLS = 32768
_NWORKERS = 32
_ROWS_PER_WORKER = _NROWS // _NWORKERS
_NSLICES = _NCOLS // 16
_UNROLL = 8


@functools.partial(
    pl.kernel,
    out_type=jax.ShapeDtypeStruct((_NROWS, 128), jnp.float32),
    scratch_types=[
        pltpu.VMEM((_NCOLS,), jnp.float32),
        pltpu.VMEM((_NCOLS,), jnp.float32),
        pltpu.VMEM((_NCOLS + 176,), jnp.int32),
        pltpu.VMEM((4368,), jnp.int32),
        pltpu.VMEM((256,), jnp.int32),
        pltpu.VMEM((128,), jnp.float32),
        pltpu.SemaphoreType.DMA,
        pltpu.SemaphoreType.DMA,
    ],
    mesh=plsc.VectorSubcoreMesh(core_axis_name="c", subcore_axis_name="s"),
    compiler_params=pltpu.CompilerParams(needs_layout_passes=False),
)
def _sc_thresholds(x_hbm, thr_hbm, row_a, row_b, cand_v, hist_v, htot_v,
                   thr_v, sem_a, sem_b):
    wid = lax.axis_index("s") * 2 + lax.axis_index("c")
    lane = lax.iota(jnp.int32, 16)
    lane_base = lane * 273
    ones16 = jnp.ones((16,), jnp.int32)
    zeros16 = jnp.zeros((16,), jnp.int32)

    def suffix(v):
        return lax.rev(jnp.cumsum(lax.rev(v, (0,))), (0,))

    def splat_count(mask):
        return jnp.max(plsc.all_reduce_population_count(mask))

    base = wid * _ROWS_PER_WORKER
    bufs = [row_a, row_b]
    sems = [sem_a, sem_b]
    handles = [None] * _ROWS_PER_WORKER
    handles[0] = pltpu.async_copy(x_hbm.at[base], row_a, sem_a)
    for r in range(_ROWS_PER_WORKER):
        row = base + r
        handles[r].wait()
        if r + 1 < _ROWS_PER_WORKER:
            handles[r + 1] = pltpu.async_copy(
                x_hbm.at[row + 1], bufs[(r + 1) % 2], sems[(r + 1) % 2])
        row_v = bufs[r % 2]

        prefix = jnp.int32(0)
        krem = jnp.int32(_K)
        ncand = jnp.int32(0)
        for p in range(4):
            shift = 24 - 8 * p

            @plsc.parallel_loop(0, 273, 1, unroll=3)
            def _(j):
                hist_v[pl.ds(j * 16, 16)] = zeros16

            pfx = prefix

            if p == 0:
                @plsc.parallel_loop(0, _NSLICES, 1, unroll=_UNROLL)
                def _(i):
                    s = i * 16
                    xv = row_v[pl.ds(s, 16)]
                    b = lax.bitcast_convert_type(xv, jnp.int32)
                    m = lax.shift_right_arithmetic(b, 31)
                    key = b ^ (m | jnp.int32(-(2**31)))
                    digit = lax.shift_right_logical(key, shift) & 255
                    plsc.addupdate_scatter(
                        hist_v, [lane_base + digit], ones16)
            elif p == 1:
                def body(i, off):
                    s = i * 16
                    xv = row_v[pl.ds(s, 16)]
                    b = lax.bitcast_convert_type(xv, jnp.int32)
                    m = lax.shift_right_arithmetic(b, 31)
                    key = b ^ (m | jnp.int32(-(2**31)))
                    act = lax.shift_right_logical(key, shift + 8) == pfx
                    pos = off + jnp.cumsum(jnp.where(act, 1, 0)) - 1
                    plsc.store_scatter(cand_v, [pos], key, mask=act)
                    return off + plsc.all_reduce_population_count(act)

                off = plsc.parallel_loop(
                    0, _NSLICES, 1, unroll=_UNROLL, carry=zeros16)(body)
                ncand = jnp.max(off)
                sent = lax.shift_left(
                    jnp.broadcast_to(pfx ^ 128, (16,)), 24)
                for t in range(10):
                    plsc.store_scatter(
                        cand_v, [off + t * 16 + lane], sent)
                nsl = jnp.max(lax.shift_right_logical(off + 15, 4))

                @plsc.parallel_loop(0, nsl, 1, unroll=4)
                def _(i):
                    s = i * 16
                    key = cand_v[pl.ds(s, 16)]
                    act = lax.shift_right_logical(key, shift + 8) == pfx
                    digit = lax.shift_right_logical(key, shift) & 255
                    plsc.addupdate_scatter(
                        hist_v, [lane_base + digit], ones16, mask=act)
            else:
                @plsc.parallel_loop(0, nsl, 1, unroll=4)
                def _(i):
                    s = i * 16
                    key = cand_v[pl.ds(s, 16)]
                    act = lax.shift_right_logical(key, shift + 8) == pfx
                    digit = lax.shift_right_logical(key, shift) & 255
                    plsc.addupdate_scatter(
                        hist_v, [lane_base + digit], ones16, mask=act)

            def merge(j, gt):
                parts = [hist_v[pl.ds(l * 273 + j * 16, 16)] for l in range(16)]
                while len(parts) > 1:
                    parts = [a + b for a, b in zip(parts[::2], parts[1::2])]
                acc = parts[0]
                htot_v[pl.ds(j * 16, 16)] = acc
                return gt + jnp.where(lane == j, jnp.sum(acc), 0)

            gt = lax.fori_loop(0, 16, merge, zeros16)
            jstar = splat_count(suffix(gt) >= krem) - 1
            hj = htot_v[pl.ds(jstar * 16, 16)]
            excl = jnp.sum(jnp.where(lane > jstar, gt, 0))
            sfx_w = suffix(hj) + excl
            lstar = splat_count(sfx_w >= krem) - 1
            s_d = excl + jnp.sum(jnp.where(lane >= lstar, hj, 0))
            h_d = jnp.sum(jnp.where(lane == lstar, hj, 0))
            krem = krem - (s_d - h_d)
            prefix = (prefix * 256) + jstar * 16 + lstar

        ukey = jnp.broadcast_to(prefix, (16,))
        thr_bits = jnp.where(ukey < 0, ukey ^ jnp.int32(-(2**31)), ~ukey)
        thr_f = lax.bitcast_convert_type(thr_bits, jnp.float32)
        for t in range(8):
            thr_v[pl.ds(t * 16, 16)] = thr_f
        pltpu.sync_copy(thr_v, thr_hbm.at[row])


def _mask_block(x_ref, t_ref, o_ref):
    x = x_ref[...]
    thr = t_ref[:, 0:1]
    o_ref[...] = jnp.where(x >= thr, x, jnp.float32(0.0))


def _tc_mask(inputs, thr_bcast):
    r = 64
    return pl.pallas_call(
        _mask_block,
        grid=(_NROWS // r,),
        in_specs=[
            pl.BlockSpec((r, _NCOLS), lambda i: (i, 0)),
            pl.BlockSpec((r, 128), lambda i: (i, 0)),
        ],
        out_specs=pl.BlockSpec((r, _NCOLS), lambda i: (i, 0)),
        out_shape=jax.ShapeDtypeStruct((_NROWS, _NCOLS), jnp.float32),
    )(inputs, thr_bcast)


def kernel(inputs, k):
    del k
    return _tc_mask(inputs, _sc_thresholds(inputs))
